# Initial kernel scaffold; baseline (speedup 1.0000x reference)
import jax
import jax.numpy as jnp
from jax.experimental import pallas as pl


def kernel(x, token_table, pos_table):
    raise NotImplementedError("write your pallas kernel here")


# speedup vs baseline: 1.4547x; 1.4547x over previous
import jax
import jax.numpy as jnp
from jax import lax
from jax.experimental import pallas as pl
from jax.experimental.pallas import tpu as pltpu
from jax.experimental.pallas import tpu_sc as plsc

_B = 4096
_T = 200
_D = 32
_N = _B * _T
_NC = 2
_NS = 16
_NW = _NC * _NS
_RPW = _N // _NW
_SEQ_PER_CHUNK = 4
_CHUNK = _SEQ_PER_CHUNK * _T
_NCHUNK = _RPW // _CHUNK
_G = 100
_NG = _CHUNK // _G
_LANES = 16


def _body(x_hbm, tab_hbm, pos_hbm, out_hbm,
          idx_a, idx_b, rows_a, rows_b, pos_v,
          sg_a, sg_b, swb_a, swb_b):
    wid = lax.axis_index("s") * _NC + lax.axis_index("c")
    base = wid * _RPW

    pltpu.sync_copy(pos_hbm, pos_v)

    idx_bufs = (idx_a, idx_b)
    rows_bufs = (rows_a, rows_b)
    sg = (sg_a, sg_b)
    swb = (swb_a, swb_b)

    def load_chunk(c, p):
        r0 = wid * (_RPW // _G) + c * _NG
        pltpu.sync_copy(x_hbm.at[pl.ds(r0, _NG)], idx_bufs[p])
        descs = []
        for g in range(_NG):
            descs.append(pltpu.async_copy(
                tab_hbm.at[idx_bufs[p].at[g]],
                rows_bufs[p].at[pl.ds(g * _G, _G)],
                sg[p]))
        return descs

    def add_pos(p):
        rv = rows_bufs[p]

        def tbody(t, carry):
            for half in range(0, _D, _LANES):
                pv = pos_v[t, pl.ds(half, _LANES)]
                for s in range(_SEQ_PER_CHUNK):
                    plsc.addupdate(rv.at[s * _T + t, pl.ds(half, _LANES)], pv)
            return carry

        lax.fori_loop(0, _T, tbody, 0)

    wb_descs = [None, None]
    g_descs = [None, None]
    g_descs[0] = load_chunk(0, 0)
    for c in range(_NCHUNK):
        p = c % 2
        q = 1 - p
        if c + 1 < _NCHUNK:
            if wb_descs[q] is not None:
                wb_descs[q].wait()
            g_descs[q] = load_chunk(c + 1, q)
        for dsc in g_descs[p]:
            dsc.wait()
        add_pos(p)
        wb_descs[p] = pltpu.async_copy(
            rows_bufs[p], out_hbm.at[pl.ds(base + c * _CHUNK, _CHUNK)], swb[p])
    wb_descs[0].wait()
    wb_descs[1].wait()


def kernel(x, token_table, pos_table):
    x2 = x.astype(jnp.int32).reshape(_N // _G, _G)
    mesh = plsc.VectorSubcoreMesh(core_axis_name="c", subcore_axis_name="s")
    out = pl.kernel(
        _body,
        out_type=jax.ShapeDtypeStruct((_N, _D), jnp.float32),
        mesh=mesh,
        compiler_params=pltpu.CompilerParams(use_tc_tiling_on_sc=False),
        scratch_types=[
            pltpu.VMEM((_NG, _G), jnp.int32),
            pltpu.VMEM((_NG, _G), jnp.int32),
            pltpu.VMEM((_CHUNK, _D), jnp.float32),
            pltpu.VMEM((_CHUNK, _D), jnp.float32),
            pltpu.VMEM((_T, _D), jnp.float32),
            pltpu.SemaphoreType.DMA,
            pltpu.SemaphoreType.DMA,
            pltpu.SemaphoreType.DMA,
            pltpu.SemaphoreType.DMA,
        ],
    )(x2, token_table, pos_table)
    return out.reshape(_B, _T, _D)
